# Initial kernel scaffold; baseline (speedup 1.0000x reference)
import jax
import jax.numpy as jnp
from jax.experimental import pallas as pl


def kernel(A, B):
    raise NotImplementedError("write your pallas kernel here")


# speedup vs baseline: 1.7043x; 1.7043x over previous
import functools

import jax
import jax.numpy as jnp
from jax.experimental import pallas as pl
from jax.experimental.pallas import tpu as pltpu

_BLK = 512


def _nn_kernel(a_ref, bt_ref, ra_ref, o_ref, m_ref, i_ref, *, n_valid, n_steps):
    k = pl.program_id(0)

    @pl.when(k == 0)
    def _init():
        m_ref[...] = jnp.full(m_ref.shape, jnp.inf, dtype=jnp.float32)
        i_ref[...] = jnp.zeros(i_ref.shape, dtype=jnp.int32)

    a = a_ref[...]
    bt = bt_ref[...]

    mm = jax.lax.dot_general(
        a, bt, (((1,), (0,)), ((), ())),
        preferred_element_type=jnp.float32)
    ra = ra_ref[...]
    rb = jnp.sum(bt * bt, axis=0, keepdims=True)
    dist = ((-2.0 * mm) + ra) + rb

    lane = jax.lax.broadcasted_iota(jnp.int32, dist.shape, 1)
    gidx = lane + k * _BLK
    valid = gidx < n_valid
    dist = jnp.where(valid, dist, jnp.inf)

    upd = dist < m_ref[...]
    m_ref[...] = jnp.where(upd, dist, m_ref[...])
    i_ref[...] = jnp.where(upd, gidx, i_ref[...])

    @pl.when(k == n_steps - 1)
    def _finish():
        m = m_ref[...]
        row_min = jnp.min(m, axis=1, keepdims=True)
        cand = jnp.where(m == row_min, i_ref[...], jnp.int32(2**31 - 1))
        o_ref[...] = jnp.min(cand, axis=1, keepdims=True)


def kernel(A, B):
    m_q, d = A.shape
    n, _ = B.shape
    n_steps = (n + _BLK - 1) // _BLK
    n_pad = n_steps * _BLK

    bt = jnp.pad(B.T, ((0, 0), (0, n_pad - n)))
    ra = jnp.sum(A ** 2, axis=-1)[:, None]

    out = pl.pallas_call(
        functools.partial(_nn_kernel, n_valid=n, n_steps=n_steps),
        grid=(n_steps,),
        in_specs=[
            pl.BlockSpec((m_q, d), lambda k: (0, 0)),
            pl.BlockSpec((d, _BLK), lambda k: (0, k)),
            pl.BlockSpec((m_q, 1), lambda k: (0, 0)),
        ],
        out_specs=pl.BlockSpec((m_q, 1), lambda k: (0, 0)),
        out_shape=jax.ShapeDtypeStruct((m_q, 1), jnp.int32),
        scratch_shapes=[
            pltpu.VMEM((m_q, _BLK), jnp.float32),
            pltpu.VMEM((m_q, _BLK), jnp.int32),
        ],
        compiler_params=pltpu.CompilerParams(
            dimension_semantics=("arbitrary",),
        ),
    )(A, bt, ra)
    return out.reshape(m_q)
